# baseline (speedup 1.0000x reference)
import functools
import math

import jax
import jax.numpy as jnp
import numpy as np
from jax import lax
from jax.experimental import pallas as pl
from jax.experimental.pallas import tpu as pltpu
from jax.experimental.pallas import tpu_sc as plsc

_L = 2048
_D = 64
_H = 16
_U = 40
_UP = 48
_KB = 512
_CB = 256
_SCALE = 1.0 / math.sqrt(_D)
_HIGH = jax.lax.Precision.HIGHEST


def _tf_rounds(x0, x1, rots):
    for r in rots:
        x0 = (x0 + x1).astype(np.uint32)
        x1 = ((x1 << np.uint32(r)) | (x1 >> np.uint32(32 - r))).astype(np.uint32)
        x1 = x0 ^ x1
    return x0, x1


def _threefry2x32(k1, k2, x1, x2):
    ks0, ks1 = np.uint32(k1), np.uint32(k2)
    ks2 = np.uint32(ks0 ^ ks1 ^ np.uint32(0x1BD11BDA))
    r0, r1 = (13, 15, 26, 6), (17, 29, 16, 24)
    x0 = (x1 + ks0).astype(np.uint32)
    y1 = (x2 + ks1).astype(np.uint32)
    for rots, ka, kb, i in ((r0, ks1, ks2, 1), (r1, ks2, ks0, 2),
                            (r0, ks0, ks1, 3), (r1, ks1, ks2, 4),
                            (r0, ks2, ks0, 5)):
        x0, y1 = _tf_rounds(x0, y1, rots)
        x0 = (x0 + ka).astype(np.uint32)
        y1 = (y1 + kb + np.uint32(i)).astype(np.uint32)
    return x0, y1


def _sample_index_np() -> np.ndarray:
    b1, b2 = _threefry2x32(np.uint32(0), np.uint32(42),
                           np.zeros(2, np.uint32), np.arange(2, dtype=np.uint32))
    n = _L * _U
    o1, o2 = _threefry2x32(np.uint32(b1[1]), np.uint32(b2[1]),
                           np.zeros(n, np.uint32), np.arange(n, dtype=np.uint32))
    bits = o1 ^ o2
    return (bits % np.uint32(_L)).astype(np.int32).reshape(_L, _U)


def _counts_t_np() -> np.ndarray:
    idx = _sample_index_np()
    cnt = np.zeros((_L, _L), np.float32)
    np.add.at(cnt, (idx.ravel(), np.repeat(np.arange(_L), _U)), 1.0)
    return cnt


_CNT_T = _counts_t_np()


def _stats_kernel(q_ref, k_ref, v_ref, cnt_ref, m_ref, o_ref):
    q = q_ref[0]
    v = v_ref[0]

    mrun = jnp.full((1, _L), -jnp.inf, jnp.float32)
    srun = jnp.zeros((1, _L), jnp.float32)
    for b in range(_L // _KB):
        kb = k_ref[0, pl.ds(b * _KB, _KB), :]
        st = jax.lax.dot_general(kb, q, (((1,), (1,)), ((), ())),
                                 preferred_element_type=jnp.float32)
        ct = cnt_ref[pl.ds(b * _KB, _KB), :]
        biased = jnp.where(ct > 0.0, st, -jnp.inf)
        mrun = jnp.maximum(mrun, jnp.max(biased, axis=0, keepdims=True))
        srun = srun + jnp.sum(ct * st, axis=0, keepdims=True)
    m_ref[0] = mrun - srun * (1.0 / _L)

    r_io = jax.lax.broadcasted_iota(jnp.int32, (_CB, _CB), 0)
    c_io = jax.lax.broadcasted_iota(jnp.int32, (_CB, _CB), 1)
    tri = (r_io >= c_io).astype(jnp.float32)
    row1 = jax.lax.broadcasted_iota(jnp.int32, (_CB, 1), 0).astype(jnp.float32)
    carry = jnp.zeros((1, _D), jnp.float32)
    for i in range(_L // _CB):
        vb = v[i * _CB:(i + 1) * _CB, :]
        cs = jnp.dot(tri, vb, preferred_element_type=jnp.float32,
                     precision=_HIGH) + carry
        o_ref[0, pl.ds(i * _CB, _CB), :] = cs / (row1 + (i * _CB + 1.0))
        carry = carry + jnp.sum(vb, axis=0, keepdims=True)


def _topk_kernel(m_ref, idx_ref, gidx_ref):
    mv = m_ref[:, 0, :]
    lane = jax.lax.broadcasted_iota(jnp.int32, (_H, _L), 1)
    head = jax.lax.broadcasted_iota(jnp.int32, (_H, 1), 0)
    gidx_ref[...] = jnp.zeros((_H, _UP), jnp.int32)
    for u in range(_U):
        mx = jnp.max(mv, axis=1, keepdims=True)
        pos = jnp.min(jnp.where(mv == mx, lane, _L), axis=1, keepdims=True)
        idx_ref[:, u:u + 1] = pos
        gidx_ref[:, u:u + 1] = (pos + head * _L) // 2
        mv = jnp.where(lane == pos, -jnp.inf, mv)


_SC_INFO = plsc.get_sparse_core_info()
_NW = _SC_INFO.num_cores * _SC_INFO.num_subcores
_BTOT = _H * _UP
_BPW = _BTOT // _NW


def _sc_gather_kernel(table_hbm, idx_hbm, out_hbm, idx_v, rows_v, sem):
    wid = lax.axis_index("s") * _SC_INFO.num_cores + lax.axis_index("c")
    base = wid * _BPW
    pltpu.sync_copy(idx_hbm.at[pl.ds(base, _BPW)], idx_v)
    pltpu.async_copy(table_hbm.at[idx_v], rows_v, sem).wait()
    pltpu.sync_copy(rows_v, out_hbm.at[pl.ds(base, _BPW)])


_sc_gather = functools.partial(
    pl.kernel,
    mesh=plsc.VectorSubcoreMesh(core_axis_name="c", subcore_axis_name="s"),
    out_type=jax.ShapeDtypeStruct((_BTOT, 2 * _D), jnp.float32),
    scratch_types=[
        pltpu.VMEM((_BPW,), jnp.int32),
        pltpu.VMEM((_BPW, 2 * _D), jnp.float32),
        pltpu.SemaphoreType.DMA,
    ],
)(_sc_gather_kernel)


def _attn_kernel(idx_ref, qr_ref, k_ref, v_ref, c_ref, o_ref):
    k = k_ref[0]
    v = v_ref[0]
    qrp = qr_ref[:_U, :]
    idxrow = idx_ref[0]

    sub = jax.lax.broadcasted_iota(jnp.int32, (_L, 1), 0)
    oht = (sub == idxrow).astype(jnp.float32)

    par_l = (sub % 2).astype(jnp.float32)
    par = jax.lax.dot_general(oht, par_l, (((0,), (0,)), ((), ())),
                              preferred_element_type=jnp.float32,
                              precision=_HIGH)
    qr = qrp[:, :_D] * (1.0 - par) + qrp[:, _D:] * par

    sc = jax.lax.dot_general(qr, k, (((1,), (1,)), ((), ())),
                             preferred_element_type=jnp.float32) * _SCALE
    sc = sc - jnp.max(sc, axis=1, keepdims=True)
    e = jnp.exp(sc)
    attn = e / jnp.sum(e, axis=1, keepdims=True)
    upd = jnp.dot(attn, v, preferred_element_type=jnp.float32)

    scattered = jnp.dot(oht, upd, preferred_element_type=jnp.float32,
                        precision=_HIGH)
    selrow = jnp.sum(oht, axis=1, keepdims=True)
    o_ref[0] = jnp.where(selrow > 0.5, scattered, c_ref[0])


@jax.jit
def _run(qh, kh, vh):
    cnt_t = jnp.asarray(_CNT_T)
    m_all, ctx0 = pl.pallas_call(
        _stats_kernel,
        grid=(_H,),
        in_specs=[
            pl.BlockSpec((1, _L, _D), lambda h: (h, 0, 0)),
            pl.BlockSpec((1, _L, _D), lambda h: (h, 0, 0)),
            pl.BlockSpec((1, _L, _D), lambda h: (h, 0, 0)),
            pl.BlockSpec((_L, _L), lambda h: (0, 0)),
        ],
        out_specs=[
            pl.BlockSpec((1, 1, _L), lambda h: (h, 0, 0)),
            pl.BlockSpec((1, _L, _D), lambda h: (h, 0, 0)),
        ],
        out_shape=[
            jax.ShapeDtypeStruct((_H, 1, _L), jnp.float32),
            jax.ShapeDtypeStruct((_H, _L, _D), jnp.float32),
        ],
    )(qh, kh, vh, cnt_t)

    idx, gidx = pl.pallas_call(
        _topk_kernel,
        out_shape=[
            jax.ShapeDtypeStruct((_H, _U), jnp.int32),
            jax.ShapeDtypeStruct((_H, _UP), jnp.int32),
        ],
    )(m_all)

    qr_all = _sc_gather(qh.reshape(_H * _L // 2, 2 * _D), gidx.reshape(_BTOT))

    idx3 = idx.reshape(_H, 1, _U)
    ctx = pl.pallas_call(
        _attn_kernel,
        grid=(_H,),
        in_specs=[
            pl.BlockSpec((1, 1, _U), lambda h: (h, 0, 0)),
            pl.BlockSpec((_UP, 2 * _D), lambda h: (h, 0)),
            pl.BlockSpec((1, _L, _D), lambda h: (h, 0, 0)),
            pl.BlockSpec((1, _L, _D), lambda h: (h, 0, 0)),
            pl.BlockSpec((1, _L, _D), lambda h: (h, 0, 0)),
        ],
        out_specs=pl.BlockSpec((1, _L, _D), lambda h: (h, 0, 0)),
        out_shape=jax.ShapeDtypeStruct((_H, _L, _D), jnp.float32),
    )(idx3, qr_all, kh, vh, ctx0)
    return ctx


def kernel(queries, keys, values, attn_mask):
    qh = jnp.transpose(queries[0], (1, 0, 2))
    kh = jnp.transpose(keys[0], (1, 0, 2))
    vh = jnp.transpose(values[0], (1, 0, 2))
    ctx = _run(qh, kh, vh)
    return jnp.transpose(ctx, (1, 0, 2))[None]


# speedup vs baseline: 1.0424x; 1.0424x over previous
import functools
import math

import jax
import jax.numpy as jnp
import numpy as np
from jax import lax
from jax.experimental import pallas as pl
from jax.experimental.pallas import tpu as pltpu
from jax.experimental.pallas import tpu_sc as plsc

_L = 2048
_D = 64
_H = 16
_U = 40
_UP = 48
_KB = 512
_CB = 256
_SCALE = 1.0 / math.sqrt(_D)
_HIGH = jax.lax.Precision.HIGHEST


def _tf_rounds(x0, x1, rots):
    for r in rots:
        x0 = (x0 + x1).astype(np.uint32)
        x1 = ((x1 << np.uint32(r)) | (x1 >> np.uint32(32 - r))).astype(np.uint32)
        x1 = x0 ^ x1
    return x0, x1


def _threefry2x32(k1, k2, x1, x2):
    ks0, ks1 = np.uint32(k1), np.uint32(k2)
    ks2 = np.uint32(ks0 ^ ks1 ^ np.uint32(0x1BD11BDA))
    r0, r1 = (13, 15, 26, 6), (17, 29, 16, 24)
    x0 = (x1 + ks0).astype(np.uint32)
    y1 = (x2 + ks1).astype(np.uint32)
    for rots, ka, kb, i in ((r0, ks1, ks2, 1), (r1, ks2, ks0, 2),
                            (r0, ks0, ks1, 3), (r1, ks1, ks2, 4),
                            (r0, ks2, ks0, 5)):
        x0, y1 = _tf_rounds(x0, y1, rots)
        x0 = (x0 + ka).astype(np.uint32)
        y1 = (y1 + kb + np.uint32(i)).astype(np.uint32)
    return x0, y1


def _sample_index_np() -> np.ndarray:
    b1, b2 = _threefry2x32(np.uint32(0), np.uint32(42),
                           np.zeros(2, np.uint32), np.arange(2, dtype=np.uint32))
    n = _L * _U
    o1, o2 = _threefry2x32(np.uint32(b1[1]), np.uint32(b2[1]),
                           np.zeros(n, np.uint32), np.arange(n, dtype=np.uint32))
    bits = o1 ^ o2
    return (bits % np.uint32(_L)).astype(np.int32).reshape(_L, _U)


def _counts_t_np() -> np.ndarray:
    idx = _sample_index_np()
    cnt = np.zeros((_L, _L), np.float32)
    np.add.at(cnt, (idx.ravel(), np.repeat(np.arange(_L), _U)), 1.0)
    return cnt


_CNT_T = _counts_t_np()


def _stats_kernel(q_ref, k_ref, v_ref, cnt_ref, m_ref, o_ref):
    r_io = jax.lax.broadcasted_iota(jnp.int32, (_CB, _CB), 0)
    c_io = jax.lax.broadcasted_iota(jnp.int32, (_CB, _CB), 1)
    tri = (r_io >= c_io).astype(jnp.float32)
    row1 = jax.lax.broadcasted_iota(jnp.int32, (_CB, 1), 0).astype(jnp.float32)

    for hh in range(2):
        lo = hh * _D
        q = q_ref[:, lo:lo + _D]
        v = v_ref[:, lo:lo + _D]

        mrun = jnp.full((1, _L), -jnp.inf, jnp.float32)
        srun = jnp.zeros((1, _L), jnp.float32)
        for b in range(_L // _KB):
            kb = k_ref[pl.ds(b * _KB, _KB), lo:lo + _D]
            st = jax.lax.dot_general(kb, q, (((1,), (1,)), ((), ())),
                                     preferred_element_type=jnp.float32)
            ct = cnt_ref[pl.ds(b * _KB, _KB), :]
            biased = jnp.where(ct > 0.0, st, -jnp.inf)
            mrun = jnp.maximum(mrun, jnp.max(biased, axis=0, keepdims=True))
            srun = srun + jnp.sum(ct * st, axis=0, keepdims=True)
        m_ref[hh] = mrun - srun * (1.0 / _L)

        carry = jnp.zeros((1, _D), jnp.float32)
        for i in range(_L // _CB):
            vb = v[i * _CB:(i + 1) * _CB, :]
            cs = jnp.dot(tri, vb, preferred_element_type=jnp.float32,
                         precision=_HIGH) + carry
            o_ref[pl.ds(i * _CB, _CB), lo:lo + _D] = cs / (row1 + (i * _CB + 1.0))
            carry = carry + jnp.sum(vb, axis=0, keepdims=True)


def _topk_kernel(m_ref, idx_ref, gidx_ref):
    mv = m_ref[:, 0, :]
    lane = jax.lax.broadcasted_iota(jnp.int32, (_H, _L), 1)
    head = jax.lax.broadcasted_iota(jnp.int32, (_H, 1), 0)
    gidx_ref[...] = jnp.zeros((_H, _UP), jnp.int32)
    for u in range(_U):
        mx = jnp.max(mv, axis=1, keepdims=True)
        pos = jnp.min(jnp.where(mv == mx, lane, _L), axis=1, keepdims=True)
        idx_ref[:, u:u + 1] = pos
        gidx_ref[:, u:u + 1] = pos * (_H // 2) + head // 2
        mv = jnp.where(lane == pos, -jnp.inf, mv)


_SC_INFO = plsc.get_sparse_core_info()
_NW = _SC_INFO.num_cores * _SC_INFO.num_subcores
_BTOT = _H * _UP
_BPW = _BTOT // _NW


def _sc_gather_kernel(table_hbm, idx_hbm, out_hbm, idx_v, rows_v, sem):
    wid = lax.axis_index("s") * _SC_INFO.num_cores + lax.axis_index("c")
    base = wid * _BPW
    pltpu.sync_copy(idx_hbm.at[pl.ds(base, _BPW)], idx_v)
    pltpu.async_copy(table_hbm.at[idx_v], rows_v, sem).wait()
    pltpu.sync_copy(rows_v, out_hbm.at[pl.ds(base, _BPW)])


_sc_gather = functools.partial(
    pl.kernel,
    mesh=plsc.VectorSubcoreMesh(core_axis_name="c", subcore_axis_name="s"),
    out_type=jax.ShapeDtypeStruct((_BTOT, 2 * _D), jnp.float32),
    scratch_types=[
        pltpu.VMEM((_BPW,), jnp.int32),
        pltpu.VMEM((_BPW, 2 * _D), jnp.float32),
        pltpu.SemaphoreType.DMA,
    ],
)(_sc_gather_kernel)


def _attn_kernel(idx_ref, qr_ref, k_ref, v_ref, c_ref, o_ref):
    sub = jax.lax.broadcasted_iota(jnp.int32, (_L, 1), 0)
    for hh in range(2):
        lo = hh * _D
        k = k_ref[:, lo:lo + _D]
        v = v_ref[:, lo:lo + _D]
        qrp = qr_ref[pl.ds(hh * _UP, _UP), :]
        qr = qrp[:_U, lo:lo + _D]
        idxrow = idx_ref[hh]

        sc = jax.lax.dot_general(qr, k, (((1,), (1,)), ((), ())),
                                 preferred_element_type=jnp.float32) * _SCALE
        sc = sc - jnp.max(sc, axis=1, keepdims=True)
        e = jnp.exp(sc)
        attn = e / jnp.sum(e, axis=1, keepdims=True)
        upd = jnp.dot(attn, v, preferred_element_type=jnp.float32)

        oht = (sub == idxrow).astype(jnp.float32)
        scattered = jnp.dot(oht, upd, preferred_element_type=jnp.float32,
                            precision=_HIGH)
        selrow = jnp.sum(oht, axis=1, keepdims=True)
        o_ref[:, lo:lo + _D] = jnp.where(selrow > 0.5, scattered,
                                         c_ref[:, lo:lo + _D])


@jax.jit
def _run(qf, kf, vf):
    cnt_t = jnp.asarray(_CNT_T)
    m_all, ctx0 = pl.pallas_call(
        _stats_kernel,
        grid=(_H // 2,),
        in_specs=[
            pl.BlockSpec((_L, 2 * _D), lambda g: (0, g)),
            pl.BlockSpec((_L, 2 * _D), lambda g: (0, g)),
            pl.BlockSpec((_L, 2 * _D), lambda g: (0, g)),
            pl.BlockSpec((_L, _L), lambda g: (0, 0)),
        ],
        out_specs=[
            pl.BlockSpec((2, 1, _L), lambda g: (g, 0, 0)),
            pl.BlockSpec((_L, 2 * _D), lambda g: (0, g)),
        ],
        out_shape=[
            jax.ShapeDtypeStruct((_H, 1, _L), jnp.float32),
            jax.ShapeDtypeStruct((_L, _H * _D), jnp.float32),
        ],
    )(qf, kf, vf, cnt_t)

    idx, gidx = pl.pallas_call(
        _topk_kernel,
        out_shape=[
            jax.ShapeDtypeStruct((_H, _U), jnp.int32),
            jax.ShapeDtypeStruct((_H, _UP), jnp.int32),
        ],
    )(m_all)

    qr_all = _sc_gather(qf.reshape(_L * _H // 2, 2 * _D), gidx.reshape(_BTOT))

    idx3 = idx.reshape(_H, 1, _U)
    ctx = pl.pallas_call(
        _attn_kernel,
        grid=(_H // 2,),
        in_specs=[
            pl.BlockSpec((2, 1, _U), lambda g: (g, 0, 0)),
            pl.BlockSpec((2 * _UP, 2 * _D), lambda g: (g, 0)),
            pl.BlockSpec((_L, 2 * _D), lambda g: (0, g)),
            pl.BlockSpec((_L, 2 * _D), lambda g: (0, g)),
            pl.BlockSpec((_L, 2 * _D), lambda g: (0, g)),
        ],
        out_specs=pl.BlockSpec((_L, 2 * _D), lambda g: (0, g)),
        out_shape=jax.ShapeDtypeStruct((_L, _H * _D), jnp.float32),
    )(idx3, qr_all, kf, vf, ctx0)
    return ctx


def kernel(queries, keys, values, attn_mask):
    qf = queries[0].reshape(_L, _H * _D)
    kf = keys[0].reshape(_L, _H * _D)
    vf = values[0].reshape(_L, _H * _D)
    ctx = _run(qf, kf, vf)
    return ctx.reshape(1, _L, _H, _D)


# speedup vs baseline: 1.0947x; 1.0501x over previous
import functools
import math

import jax
import jax.numpy as jnp
import numpy as np
from jax import lax
from jax.experimental import pallas as pl
from jax.experimental.pallas import tpu as pltpu
from jax.experimental.pallas import tpu_sc as plsc

_L = 2048
_D = 64
_H = 16
_U = 40
_UP = 48
_KB = 512
_CB = 256
_SCALE = 1.0 / math.sqrt(_D)
_HIGH = jax.lax.Precision.HIGHEST


def _tf_rounds(x0, x1, rots):
    for r in rots:
        x0 = (x0 + x1).astype(np.uint32)
        x1 = ((x1 << np.uint32(r)) | (x1 >> np.uint32(32 - r))).astype(np.uint32)
        x1 = x0 ^ x1
    return x0, x1


def _threefry2x32(k1, k2, x1, x2):
    ks0, ks1 = np.uint32(k1), np.uint32(k2)
    ks2 = np.uint32(ks0 ^ ks1 ^ np.uint32(0x1BD11BDA))
    r0, r1 = (13, 15, 26, 6), (17, 29, 16, 24)
    x0 = (x1 + ks0).astype(np.uint32)
    y1 = (x2 + ks1).astype(np.uint32)
    for rots, ka, kb, i in ((r0, ks1, ks2, 1), (r1, ks2, ks0, 2),
                            (r0, ks0, ks1, 3), (r1, ks1, ks2, 4),
                            (r0, ks2, ks0, 5)):
        x0, y1 = _tf_rounds(x0, y1, rots)
        x0 = (x0 + ka).astype(np.uint32)
        y1 = (y1 + kb + np.uint32(i)).astype(np.uint32)
    return x0, y1


def _sample_index_np() -> np.ndarray:
    b1, b2 = _threefry2x32(np.uint32(0), np.uint32(42),
                           np.zeros(2, np.uint32), np.arange(2, dtype=np.uint32))
    n = _L * _U
    o1, o2 = _threefry2x32(np.uint32(b1[1]), np.uint32(b2[1]),
                           np.zeros(n, np.uint32), np.arange(n, dtype=np.uint32))
    bits = o1 ^ o2
    return (bits % np.uint32(_L)).astype(np.int32).reshape(_L, _U)


def _counts_t_np() -> np.ndarray:
    idx = _sample_index_np()
    cnt = np.zeros((_L, _L), np.float32)
    np.add.at(cnt, (idx.ravel(), np.repeat(np.arange(_L), _U)), 1.0)
    return cnt


_CNT_T = _counts_t_np()
_NEG_BIAS = np.where(_CNT_T > 0.0, np.float32(0.0),
                     np.float32(-np.inf)).astype(np.float32)


def _stats_kernel(q_ref, k_ref, cnt_ref, bias_ref, m_ref):
    for hh in range(2):
        lo = hh * _D
        q = q_ref[:, lo:lo + _D]

        mrun = jnp.full((1, _L), -jnp.inf, jnp.float32)
        srun = jnp.zeros((1, _L), jnp.float32)
        for b in range(_L // _KB):
            kb = k_ref[pl.ds(b * _KB, _KB), lo:lo + _D]
            st = jax.lax.dot_general(kb, q, (((1,), (1,)), ((), ())),
                                     preferred_element_type=jnp.float32)
            ct = cnt_ref[pl.ds(b * _KB, _KB), :]
            biased = st + bias_ref[pl.ds(b * _KB, _KB), :]
            mrun = jnp.maximum(mrun, jnp.max(biased, axis=0, keepdims=True))
            srun = srun + jnp.sum(ct * st, axis=0, keepdims=True)
        m_ref[hh] = mrun - srun * (1.0 / _L)


def _topk_kernel(m_ref, idx_ref, gidx_ref):
    mv = m_ref[:, 0, :]
    lane = jax.lax.broadcasted_iota(jnp.int32, (_H, _L), 1)
    head = jax.lax.broadcasted_iota(jnp.int32, (_H, 1), 0)
    gidx_ref[...] = jnp.zeros((_H, _UP), jnp.int32)
    for u in range(_U):
        mx = jnp.max(mv, axis=1, keepdims=True)
        pos = jnp.min(jnp.where(mv == mx, lane, _L), axis=1, keepdims=True)
        idx_ref[:, u:u + 1] = pos
        gidx_ref[:, u:u + 1] = pos * (_H // 2) + head // 2
        mv = jnp.where(lane == pos, -jnp.inf, mv)


_SC_INFO = plsc.get_sparse_core_info()
_NW = _SC_INFO.num_cores * _SC_INFO.num_subcores
_BTOT = _H * _UP
_BPW = _BTOT // _NW


def _sc_gather_kernel(table_hbm, idx_hbm, out_hbm, idx_v, rows_v, sem):
    wid = lax.axis_index("s") * _SC_INFO.num_cores + lax.axis_index("c")
    base = wid * _BPW
    pltpu.sync_copy(idx_hbm.at[pl.ds(base, _BPW)], idx_v)
    pltpu.async_copy(table_hbm.at[idx_v], rows_v, sem).wait()
    pltpu.sync_copy(rows_v, out_hbm.at[pl.ds(base, _BPW)])


_sc_gather = functools.partial(
    pl.kernel,
    mesh=plsc.VectorSubcoreMesh(core_axis_name="c", subcore_axis_name="s"),
    out_type=jax.ShapeDtypeStruct((_BTOT, 2 * _D), jnp.float32),
    scratch_types=[
        pltpu.VMEM((_BPW,), jnp.int32),
        pltpu.VMEM((_BPW, 2 * _D), jnp.float32),
        pltpu.SemaphoreType.DMA,
    ],
)(_sc_gather_kernel)


def _attn_kernel(idx_ref, qr_ref, k_ref, v_ref, o_ref):
    sub = jax.lax.broadcasted_iota(jnp.int32, (_L, 1), 0)
    r_io = jax.lax.broadcasted_iota(jnp.int32, (_CB, _CB), 0)
    c_io = jax.lax.broadcasted_iota(jnp.int32, (_CB, _CB), 1)
    tri = (r_io >= c_io).astype(jnp.float32)
    row1 = jax.lax.broadcasted_iota(jnp.int32, (_CB, 1), 0).astype(jnp.float32)
    for hh in range(2):
        lo = hh * _D
        k = k_ref[:, lo:lo + _D]
        v = v_ref[:, lo:lo + _D]
        qrp = qr_ref[pl.ds(hh * _UP, _UP), :]
        qr = qrp[:_U, lo:lo + _D]
        idxrow = idx_ref[hh]

        sc = jax.lax.dot_general(qr, k, (((1,), (1,)), ((), ())),
                                 preferred_element_type=jnp.float32) * _SCALE
        sc = sc - jnp.max(sc, axis=1, keepdims=True)
        e = jnp.exp(sc)
        attn = e / jnp.sum(e, axis=1, keepdims=True)
        upd = jnp.dot(attn, v, preferred_element_type=jnp.float32)

        oht = (sub == idxrow).astype(jnp.float32)
        scattered = jnp.dot(oht, upd, preferred_element_type=jnp.float32,
                            precision=_HIGH)
        selrow = jnp.sum(oht, axis=1, keepdims=True)

        carry = jnp.zeros((1, _D), jnp.float32)
        for i in range(_L // _CB):
            s = slice(i * _CB, (i + 1) * _CB)
            vb = v[s, :]
            cs = jnp.dot(tri, vb, preferred_element_type=jnp.float32,
                         precision=_HIGH) + carry
            cs = cs / (row1 + (i * _CB + 1.0))
            o_ref[pl.ds(i * _CB, _CB), lo:lo + _D] = jnp.where(
                selrow[s, :] > 0.5, scattered[s, :], cs)
            carry = carry + jnp.sum(vb, axis=0, keepdims=True)


@jax.jit
def _run(qf, kf, vf):
    cnt_t = jnp.asarray(_CNT_T)
    bias_t = jnp.asarray(_NEG_BIAS)
    m_all = pl.pallas_call(
        _stats_kernel,
        grid=(_H // 2,),
        in_specs=[
            pl.BlockSpec((_L, 2 * _D), lambda g: (0, g)),
            pl.BlockSpec((_L, 2 * _D), lambda g: (0, g)),
            pl.BlockSpec((_L, _L), lambda g: (0, 0)),
            pl.BlockSpec((_L, _L), lambda g: (0, 0)),
        ],
        out_specs=pl.BlockSpec((2, 1, _L), lambda g: (g, 0, 0)),
        out_shape=jax.ShapeDtypeStruct((_H, 1, _L), jnp.float32),
    )(qf, kf, cnt_t, bias_t)

    idx, gidx = pl.pallas_call(
        _topk_kernel,
        out_shape=[
            jax.ShapeDtypeStruct((_H, _U), jnp.int32),
            jax.ShapeDtypeStruct((_H, _UP), jnp.int32),
        ],
    )(m_all)

    qr_all = _sc_gather(qf.reshape(_L * _H // 2, 2 * _D), gidx.reshape(_BTOT))

    idx3 = idx.reshape(_H, 1, _U)
    ctx = pl.pallas_call(
        _attn_kernel,
        grid=(_H // 2,),
        in_specs=[
            pl.BlockSpec((2, 1, _U), lambda g: (g, 0, 0)),
            pl.BlockSpec((2 * _UP, 2 * _D), lambda g: (g, 0)),
            pl.BlockSpec((_L, 2 * _D), lambda g: (0, g)),
            pl.BlockSpec((_L, 2 * _D), lambda g: (0, g)),
        ],
        out_specs=pl.BlockSpec((_L, 2 * _D), lambda g: (0, g)),
        out_shape=jax.ShapeDtypeStruct((_L, _H * _D), jnp.float32),
    )(idx3, qr_all, kf, vf)
    return ctx


def kernel(queries, keys, values, attn_mask):
    qf = queries[0].reshape(_L, _H * _D)
    kf = keys[0].reshape(_L, _H * _D)
    vf = values[0].reshape(_L, _H * _D)
    ctx = _run(qf, kf, vf)
    return ctx.reshape(1, _L, _H, _D)
